# baseline (speedup 1.0000x reference)
import jax
import jax.numpy as jnp
from jax.experimental import pallas as pl
from jax.experimental.pallas import tpu as pltpu

_CHUNK = 1024
_DEPTH = 4


def _body(x_hbm, w1_hbm, thr_hbm, w2_hbm, o_ref, x_buf, w1_ref, thr_ref, w2_ref, sems, wsem):
    n_chunks = x_hbm.shape[0] // _CHUNK

    w1_cp = pltpu.make_async_copy(w1_hbm, w1_ref, wsem.at[0])
    thr_cp = pltpu.make_async_copy(thr_hbm, thr_ref, wsem.at[1])
    w2_cp = pltpu.make_async_copy(w2_hbm, w2_ref, wsem.at[2])
    w1_cp.start()
    thr_cp.start()
    w2_cp.start()

    def copy(i, slot):
        return pltpu.make_async_copy(
            x_hbm.at[pl.ds(i * _CHUNK, _CHUNK), :],
            x_buf.at[slot],
            sems.at[slot],
        )

    for j in range(_DEPTH):
        copy(j, j).start()

    w1_cp.wait()
    thr_cp.wait()
    w2_cp.wait()
    w1 = w1_ref[:]
    thr = thr_ref[:]
    w2 = w2_ref[:]
    for i in range(n_chunks):
        slot = i % _DEPTH
        copy(i, slot).wait()
        u = jnp.dot(x_buf[slot], w1, preferred_element_type=jnp.float32)
        h = jnp.tanh(u - thr)
        o_ref[pl.ds(i * _CHUNK, _CHUNK), :] = jnp.tanh(
            jnp.dot(h, w2, preferred_element_type=jnp.float32) - 0.5
        )
        if i + _DEPTH < n_chunks:
            copy(i + _DEPTH, slot).start()


def kernel(x, W1, thr_h, W2):
    batch, in_size = x.shape
    hidden = W1.shape[1]
    out_size = W2.shape[1]
    thr2d = thr_h.reshape(1, hidden)

    return pl.pallas_call(
        _body,
        in_specs=[
            pl.BlockSpec(memory_space=pl.ANY),
            pl.BlockSpec(memory_space=pl.ANY),
            pl.BlockSpec(memory_space=pl.ANY),
            pl.BlockSpec(memory_space=pl.ANY),
        ],
        out_specs=pl.BlockSpec(memory_space=pltpu.VMEM),
        out_shape=jax.ShapeDtypeStruct((batch, out_size), jnp.float32),
        scratch_shapes=[
            pltpu.VMEM((_DEPTH, _CHUNK, in_size), jnp.float32),
            pltpu.VMEM((in_size, hidden), jnp.float32),
            pltpu.VMEM((1, hidden), jnp.float32),
            pltpu.VMEM((hidden, out_size), jnp.float32),
            pltpu.SemaphoreType.DMA((_DEPTH,)),
            pltpu.SemaphoreType.DMA((3,)),
        ],
    )(x, W1, thr2d, W2)


# speedup vs baseline: 1.0487x; 1.0487x over previous
import jax
import jax.numpy as jnp
from jax.experimental import pallas as pl
from jax.experimental.pallas import tpu as pltpu

_CHUNK = 1024
_DEPTH = 4


def _body(x_hbm, p_hbm, o_hbm, x_buf, p_ref, o_stage, sems, psem, osems):
    n_chunks = x_hbm.shape[0] // _CHUNK

    p_cp = pltpu.make_async_copy(p_hbm, p_ref, psem)
    p_cp.start()

    def copy(i, slot):
        return pltpu.make_async_copy(
            x_hbm.at[pl.ds(i * _CHUNK, _CHUNK), :],
            x_buf.at[slot],
            sems.at[slot],
        )

    def out_copy(i, slot):
        return pltpu.make_async_copy(
            o_stage.at[slot],
            o_hbm.at[pl.ds(i * _CHUNK, _CHUNK), :],
            osems.at[slot],
        )

    for j in range(_DEPTH):
        copy(j, j).start()

    p_cp.wait()
    w1 = p_ref[:512, :8]
    thr = p_ref[512:513, :8]
    w2 = p_ref[520:528, :4]
    for i in range(n_chunks):
        slot = i % _DEPTH
        copy(i, slot).wait()
        u = jnp.dot(x_buf[slot], w1, preferred_element_type=jnp.float32)
        h = jnp.tanh(u - thr)
        o = jnp.tanh(jnp.dot(h, w2, preferred_element_type=jnp.float32) - 0.5)
        if i >= _DEPTH:
            out_copy(i - _DEPTH, slot).wait()
        o_stage[slot] = o
        out_copy(i, slot).start()
        if i + _DEPTH < n_chunks:
            copy(i + _DEPTH, slot).start()
    for i in range(n_chunks - _DEPTH, n_chunks):
        out_copy(i, i % _DEPTH).wait()


def kernel(x, W1, thr_h, W2):
    batch, in_size = x.shape
    hidden = W1.shape[1]
    out_size = W2.shape[1]

    packed = jnp.zeros((528, 128), jnp.float32)
    packed = packed.at[:512, :hidden].set(W1)
    packed = packed.at[512, :hidden].set(thr_h)
    packed = packed.at[520:528, :out_size].set(W2)

    return pl.pallas_call(
        _body,
        in_specs=[
            pl.BlockSpec(memory_space=pl.ANY),
            pl.BlockSpec(memory_space=pl.ANY),
        ],
        out_specs=pl.BlockSpec(memory_space=pl.ANY),
        out_shape=jax.ShapeDtypeStruct((batch, out_size), jnp.float32),
        scratch_shapes=[
            pltpu.VMEM((_DEPTH, _CHUNK, in_size), jnp.float32),
            pltpu.VMEM((528, 128), jnp.float32),
            pltpu.VMEM((_DEPTH, _CHUNK, out_size), jnp.float32),
            pltpu.SemaphoreType.DMA((_DEPTH,)),
            pltpu.SemaphoreType.DMA,
            pltpu.SemaphoreType.DMA((_DEPTH,)),
        ],
    )(x, packed)


# speedup vs baseline: 1.5469x; 1.4751x over previous
import jax
import jax.numpy as jnp
from jax import lax
from jax.experimental import pallas as pl
from jax.experimental.pallas import tpu as pltpu

_CHUNK = 1024
_DEPTH = 4


def _body(x_hbm, p_hbm, o_ref, x_buf, p_ref, sems, psem):
    n_chunks = x_hbm.shape[0] // _CHUNK

    p_cp = pltpu.make_async_copy(p_hbm, p_ref, psem)
    p_cp.start()

    def copy(i, slot):
        return pltpu.make_async_copy(
            x_hbm.at[pl.ds(i * _CHUNK, _CHUNK), :],
            x_buf.at[slot],
            sems.at[slot],
        )

    for j in range(_DEPTH):
        copy(j, j).start()

    p_cp.wait()
    w1t = p_ref[0:8, :]
    w2t = p_ref[8:16, 0:8]
    thr_col = p_ref[16:24, 0:1]
    for i in range(n_chunks):
        slot = i % _DEPTH
        copy(i, slot).wait()
        ut = lax.dot_general(
            w1t, x_buf[slot],
            (((1,), (1,)), ((), ())),
            preferred_element_type=jnp.float32,
        )
        ht = jnp.tanh(ut - thr_col)
        ot = jnp.tanh(
            lax.dot_general(
                w2t, ht,
                (((1,), (0,)), ((), ())),
                preferred_element_type=jnp.float32,
            )
            - 0.5
        )
        o_ref[:, pl.ds(i * _CHUNK, _CHUNK)] = ot
        if i + _DEPTH < n_chunks:
            copy(i + _DEPTH, slot).start()


def kernel(x, W1, thr_h, W2):
    batch, in_size = x.shape
    hidden = W1.shape[1]
    out_size = W2.shape[1]

    packed = jnp.zeros((24, 512), jnp.float32)
    packed = packed.at[0:hidden, :].set(W1.T)
    packed = packed.at[8:8 + out_size, :hidden].set(W2.T)
    packed = packed.at[16:16 + hidden, 0].set(thr_h)

    res_t = pl.pallas_call(
        _body,
        in_specs=[
            pl.BlockSpec(memory_space=pl.ANY),
            pl.BlockSpec(memory_space=pl.ANY),
        ],
        out_specs=pl.BlockSpec(memory_space=pltpu.VMEM),
        out_shape=jax.ShapeDtypeStruct((8, batch), jnp.float32),
        scratch_shapes=[
            pltpu.VMEM((_DEPTH, _CHUNK, in_size), jnp.float32),
            pltpu.VMEM((24, 512), jnp.float32),
            pltpu.SemaphoreType.DMA((_DEPTH,)),
            pltpu.SemaphoreType.DMA,
        ],
    )(x, packed)
    return res_t[:out_size].T
